# baseline (speedup 1.0000x reference)
import jax
import jax.numpy as jnp
from jax import lax
from jax.experimental import pallas as pl
from jax.experimental.pallas import tpu as pltpu
from jax.experimental.pallas import tpu_sc as plsc

DIM = 2048
UNIT_DIM = 64
N = 8192
BATCH = 16384
N_CHUNKS = DIM // UNIT_DIM
N_BLOCKS = DIM // 128

NUM_CORES = 2
NUM_SUBCORES = 16
NW = NUM_CORES * NUM_SUBCORES
S_PER_W = BATCH // NW
K = 128
ST_PER_M = S_PER_W // K
NSLOT = N_BLOCKS * ST_PER_M
NBUF = 2


NA = 3
NO = 2


def _gather_body(in_hbm, ge_hbm, go_hbm, out_hbm, ide_v, ido_v, a_v, o_v,
                 *sems):
    gsa = sems[0:NA]
    gso = sems[NA:NA + NO]
    ssc = sems[NA + NO:NA + NO + NA]
    wid = lax.axis_index("s") * NUM_CORES + lax.axis_index("c")
    s_base = wid * S_PER_W
    pltpu.sync_copy(ge_hbm.at[wid], ide_v)
    pltpu.sync_copy(go_hbm.at[wid], ido_v)

    def col_ref(j):
        m = j // ST_PER_M
        return in_hbm.at[:, pl.ds(m * 128, 128)]

    def start_gathers(j, ba, bo):
        pltpu.async_copy(col_ref(j).at[ide_v.at[j]], a_v.at[ba], gsa[ba])
        pltpu.async_copy(col_ref(j).at[ido_v.at[j]], o_v.at[bo], gso[bo])

    def out_slice(j):
        m, st = j // ST_PER_M, j % ST_PER_M
        return out_hbm.at[pl.ds(s_base + st * K, K), pl.ds(m * 128, 128)]

    def wait_scatter(j, ba):
        pltpu.make_async_copy(a_v.at[ba], out_slice(j), ssc[ba]).wait()

    def do_slot(j, ba, bo, bna, bno, wait_prev, start_next):
        pltpu.make_async_copy(col_ref(j).at[ide_v.at[j]], a_v.at[ba],
                              gsa[ba]).wait()
        pltpu.make_async_copy(col_ref(j).at[ido_v.at[j]], o_v.at[bo],
                              gso[bo]).wait()

        def patch(i, c):
            for u in range(4):
                for k in range(4):
                    a_v[ba, 4 * i + u, pl.ds(64 + 16 * k, 16)] = (
                        o_v[bo, 4 * i + u, pl.ds(64 + 16 * k, 16)])
            return c

        lax.fori_loop(0, K // 4, patch, 0)
        pltpu.async_copy(a_v.at[ba], out_slice(j), ssc[ba])
        if wait_prev:
            wait_scatter(j - 1, bna)
        if start_next:
            start_gathers(j + 2, bna, bno)

    start_gathers(0, 0, 0)
    start_gathers(1, 1, 1)
    do_slot(0, 0, 0, 2, 0, False, True)

    def round_body(r, carry):
        for b in range(6):
            j = 1 + r * 6 + b
            do_slot(j, (1 + b) % NA, (1 + b) % NO, b % NA, (1 + b) % NO,
                    True, True)
        return carry

    lax.fori_loop(0, (NSLOT - 4) // 6, round_body, 0)
    for j in (NSLOT - 3, NSLOT - 2, NSLOT - 1):
        do_slot(j, j % NA, j % NO, (j - 1) % NA, j % NO, True,
                j + 2 < NSLOT)
    wait_scatter(NSLOT - 1, (NSLOT - 1) % NA)


@jax.jit
def _sc_gather(inputs, ge, go):
    mesh = plsc.VectorSubcoreMesh(core_axis_name="c", subcore_axis_name="s")
    return pl.kernel(
        _gather_body,
        out_type=jax.ShapeDtypeStruct((BATCH, DIM), jnp.float32),
        mesh=mesh,
        scratch_types=[
            pltpu.VMEM((NSLOT, K), jnp.int32),
            pltpu.VMEM((NSLOT, K), jnp.int32),
            pltpu.VMEM((NA, K, 128), jnp.float32),
            pltpu.VMEM((NO, K, 128), jnp.float32),
        ] + [pltpu.SemaphoreType.DMA] * (2 * NA + NO),
        compiler_params=pltpu.CompilerParams(use_tc_tiling_on_sc=True),
    )(inputs, ge, go)


def kernel(inputs):
    idx_key = jax.random.key(1)
    keys = jax.vmap(lambda i: jax.random.fold_in(idx_key, i))(jnp.arange(N_CHUNKS))
    idx = jax.vmap(lambda k: jax.random.randint(k, (BATCH,), 0, N))(keys)

    def arrange(vh):
        g = vh.reshape(N_BLOCKS, NW, ST_PER_M, K).transpose(1, 0, 2, 3)
        return g.reshape(NW, NSLOT, K)

    ge = arrange(idx[0::2])
    go = arrange(idx[1::2])
    return _sc_gather(inputs, ge, go)


# speedup vs baseline: 1.0782x; 1.0782x over previous
import jax
import jax.numpy as jnp
from jax import lax
from jax.experimental import pallas as pl
from jax.experimental.pallas import tpu as pltpu
from jax.experimental.pallas import tpu_sc as plsc

DIM = 2048
UNIT_DIM = 64
N = 8192
BATCH = 16384
N_CHUNKS = DIM // UNIT_DIM
N_BLOCKS = DIM // 128

NUM_CORES = 2
NUM_SUBCORES = 16
NW = NUM_CORES * NUM_SUBCORES
S_PER_W = BATCH // NW
K = 128
ST_PER_M = S_PER_W // K
NSLOT = N_BLOCKS * ST_PER_M
NBUF = 2


def _gather_body(in_hbm, idx_hbm, out_hbm, idx_v, a_v, o_v, *sems):
    gse = sems[0:NBUF]
    gso = sems[NBUF:2 * NBUF]
    ssc = sems[2 * NBUF:3 * NBUF]
    wid = lax.axis_index("s") * NUM_CORES + lax.axis_index("c")
    s_base = wid * S_PER_W
    pltpu.sync_copy(idx_hbm.at[:, pl.ds(wid * ST_PER_M, ST_PER_M)], idx_v)

    def col_ref(j):
        m = j // ST_PER_M
        return in_hbm.at[:, pl.ds(m * 128, 128)]

    def idx_slices(j):
        m, st = j // ST_PER_M, j % ST_PER_M
        return idx_v.at[2 * m, st], idx_v.at[2 * m + 1, st]

    def start_gathers(j, b):
        ie, io = idx_slices(j)
        pltpu.async_copy(col_ref(j).at[ie], a_v.at[b], gse[b])
        pltpu.async_copy(col_ref(j).at[io], o_v.at[b], gso[b])

    def out_slice(j):
        m, st = j // ST_PER_M, j % ST_PER_M
        return out_hbm.at[pl.ds(s_base + st * K, K), pl.ds(m * 128, 128)]

    def do_slot(j, b, start_next):
        ie, io = idx_slices(j)
        pltpu.make_async_copy(col_ref(j).at[ie], a_v.at[b], gse[b]).wait()
        pltpu.make_async_copy(col_ref(j).at[io], o_v.at[b], gso[b]).wait()

        def patch(i, c):
            for u in range(4):
                for k in range(4):
                    a_v[b, 4 * i + u, pl.ds(64 + 16 * k, 16)] = (
                        o_v[b, 4 * i + u, pl.ds(64 + 16 * k, 16)])
            return c

        lax.fori_loop(0, K // 4, patch, 0)
        pltpu.async_copy(a_v.at[b], out_slice(j), ssc[b])
        pltpu.make_async_copy(a_v.at[b], out_slice(j), ssc[b]).wait()
        if start_next:
            start_gathers(j + NBUF, b)

    for b in range(NBUF):
        start_gathers(b, b)

    def round_body(r, carry):
        for b in range(NBUF):
            do_slot(r * NBUF + b, b, True)
        return carry

    nrounds = NSLOT // NBUF
    lax.fori_loop(0, nrounds - 1, round_body, 0)
    for b in range(NBUF):
        do_slot((nrounds - 1) * NBUF + b, b, False)


@jax.jit
def _sc_gather(inputs, idxr):
    mesh = plsc.VectorSubcoreMesh(core_axis_name="c", subcore_axis_name="s")
    return pl.kernel(
        _gather_body,
        out_type=jax.ShapeDtypeStruct((BATCH, DIM), jnp.float32),
        mesh=mesh,
        scratch_types=[
            pltpu.VMEM((N_CHUNKS, ST_PER_M, K), jnp.int32),
            pltpu.VMEM((NBUF, K, 128), jnp.float32),
            pltpu.VMEM((NBUF, K, 128), jnp.float32),
        ] + [pltpu.SemaphoreType.DMA] * (3 * NBUF),
        compiler_params=pltpu.CompilerParams(use_tc_tiling_on_sc=True),
    )(inputs, idxr)


def kernel(inputs):
    idx_key = jax.random.key(1)
    keys = jax.vmap(lambda i: jax.random.fold_in(idx_key, i))(jnp.arange(N_CHUNKS))
    idx = jax.vmap(lambda k: jax.random.randint(k, (BATCH,), 0, N))(keys)
    idxr = idx.reshape(N_CHUNKS, BATCH // K, K)
    return _sc_gather(inputs, idxr)
